# baseline (speedup 1.0000x reference)
import jax
import jax.numpy as jnp
from jax import lax
from jax.experimental import pallas as pl

N = 10
E = 30
F = 4
NF = N * F
H = 24
TL = 32768


def _prep_kernel(ei_ref, w1_ref, w2_ref, w3_ref, b1_ref, b2_ref, b3_ref,
                 wl1_ref, bl1_ref, wl2_ref,
                 m1t_ref, m2t_ref, m3t_ref, b1c_ref, b2c_ref, b3c_ref,
                 wl1t_ref, bl1c_ref, wl2t_ref):
    f32 = jnp.float32
    cdim = lambda a, b: (((a,), (b,)), ((), ()))
    dg = lambda a, b, c: lax.dot_general(a, b, c, preferred_element_type=f32)
    dot = lambda a, b: jnp.dot(a, b, preferred_element_type=f32)

    ei = ei_ref[...]
    ei0 = ei[0:1, :]
    ei1 = ei[1:2, :]
    niota = lax.broadcasted_iota(jnp.int32, (N, E), 0)
    ST = (ei0 == niota).astype(f32)
    DT = (ei1 == niota).astype(f32)

    deg = jnp.sum(DT, axis=1, keepdims=True) + 1.0
    dinv = lax.rsqrt(deg)

    dinv_src = dg(dinv, ST, cdim(0, 0))
    dinv_dst = dg(dinv, DT, cdim(0, 0))
    norm = dinv_src * dinv_dst

    AT = dg(ST * norm, DT, cdim(1, 1))
    ii = lax.broadcasted_iota(jnp.int32, (N, N), 0)
    jj = lax.broadcasted_iota(jnp.int32, (N, N), 1)
    AT = AT + jnp.where(ii == jj, dinv * dinv, 0.0)

    mi = lax.broadcasted_iota(jnp.int32, (N, NF), 0)
    ji = lax.broadcasted_iota(jnp.int32, (N, NF), 1)
    Ecol = (ji // F == mi).astype(f32)
    fi = lax.broadcasted_iota(jnp.int32, (F, NF), 0)
    gi = lax.broadcasted_iota(jnp.int32, (F, NF), 1)
    T4 = (gi % F == fi).astype(f32)

    AtE = dg(Ecol, AT, cdim(0, 1))
    w1c = dg(T4, w1_ref[...], cdim(0, 1))
    m1t_ref[...] = AtE * w1c

    R = dg(Ecol, AT, cdim(0, 1))
    ATeeT = dot(R, Ecol)
    U2 = dg(T4, w2_ref[...], cdim(0, 1))
    m2t_ref[...] = ATeeT * dot(U2, T4)
    U3 = dg(T4, w3_ref[...], cdim(0, 1))
    m3t_ref[...] = ATeeT * dot(U3, T4)

    b1c_ref[...] = dg(T4, b1_ref[...], cdim(0, 1))
    b2c_ref[...] = dg(T4, b2_ref[...], cdim(0, 1))
    b3c_ref[...] = dg(T4, b3_ref[...], cdim(0, 1))

    i40a = lax.broadcasted_iota(jnp.int32, (NF, NF), 0)
    i40b = lax.broadcasted_iota(jnp.int32, (NF, NF), 1)
    I40 = (i40a == i40b).astype(f32)
    wl1t_ref[...] = dg(wl1_ref[...], I40, cdim(0, 0))
    i24a = lax.broadcasted_iota(jnp.int32, (H, H), 0)
    i24b = lax.broadcasted_iota(jnp.int32, (H, H), 1)
    I24 = (i24a == i24b).astype(f32)
    bl1c_ref[...] = dg(I24, bl1_ref[...], cdim(0, 1))
    wl2t_ref[...] = dg(wl2_ref[...], I24, cdim(0, 0))


def _chain_kernel(x_ref, m1t_ref, m2t_ref, m3t_ref, b1c_ref, b2c_ref,
                  b3c_ref, wl1t_ref, bl1c_ref, wl2t_ref, bl2_ref, out_ref):
    dot = lambda a, b: jnp.dot(a, b, preferred_element_type=jnp.float32)
    xT = x_ref[...]
    h1 = jnp.maximum(dot(m1t_ref[...], xT) + b1c_ref[...], 0.0)
    h2 = jnp.maximum(dot(m2t_ref[...], h1) + b2c_ref[...], 0.0)
    h3 = jnp.maximum(dot(m3t_ref[...], h2) + b3c_ref[...] + h1, 0.0)
    z = jnp.maximum(dot(wl1t_ref[...], h3) + bl1c_ref[...], 0.0)
    out_ref[...] = dot(wl2t_ref[...], z) + bl2_ref[...]


def kernel(x1, edge_index, W1, b1, W2, b2, W3, b3, Wl1, bl1, Wl2, bl2):
    B = x1.shape[0]
    ei = edge_index.astype(jnp.int32)

    whole = lambda *shape: pl.BlockSpec(shape, lambda: tuple(0 for _ in shape))
    f32 = jnp.float32
    sds = jax.ShapeDtypeStruct
    (M1T, M2T, M3T, b1c, b2c, b3c, Wl1T, bl1c, Wl2T) = pl.pallas_call(
        _prep_kernel,
        in_specs=[whole(2, E), whole(1, F), whole(F, F), whole(F, F),
                  whole(1, F), whole(1, F), whole(1, F),
                  whole(NF, H), whole(1, H), whole(H, 1)],
        out_specs=[whole(NF, N), whole(NF, NF), whole(NF, NF),
                   whole(NF, 1), whole(NF, 1), whole(NF, 1),
                   whole(H, NF), whole(H, 1), whole(1, H)],
        out_shape=[sds((NF, N), f32), sds((NF, NF), f32), sds((NF, NF), f32),
                   sds((NF, 1), f32), sds((NF, 1), f32), sds((NF, 1), f32),
                   sds((H, NF), f32), sds((H, 1), f32), sds((1, H), f32)],
    )(ei, W1, W2, W3, b1[None, :], b2[None, :], b3[None, :],
      Wl1, bl1[None, :], Wl2)

    xT = x1.reshape(B, N).T
    full = lambda shape: pl.BlockSpec(shape, lambda i: (0, 0))
    outT = pl.pallas_call(
        _chain_kernel,
        grid=(B // TL,),
        in_specs=[
            pl.BlockSpec((N, TL), lambda i: (0, i)),
            full((NF, N)), full((NF, NF)), full((NF, NF)),
            full((NF, 1)), full((NF, 1)), full((NF, 1)),
            full((H, NF)), full((H, 1)), full((1, H)), full((1, 1)),
        ],
        out_specs=pl.BlockSpec((1, TL), lambda i: (0, i)),
        out_shape=sds((1, B), f32),
    )(xT, M1T, M2T, M3T, b1c, b2c, b3c, Wl1T, bl1c, Wl2T, bl2[None, :])
    return outT.reshape(B, 1)


# speedup vs baseline: 1.0852x; 1.0852x over previous
import jax
import jax.numpy as jnp
from jax import lax
from jax.experimental import pallas as pl
from jax.experimental.pallas import tpu as pltpu

N = 10
E = 30
F = 4
NF = N * F
H = 24
TL = 16384


def _fused_kernel(ei_ref, w1_ref, w2_ref, w3_ref, b1_ref, b2_ref, b3_ref,
                  wl1_ref, bl1_ref, wl2_ref, bl2_ref, x_ref, out_ref,
                  m1t_ref, m2t_ref, m3t_ref, b123c_ref,
                  wl1t_ref, blc_ref, wl2t_ref):
    f32 = jnp.float32
    cdim = lambda a, b: (((a,), (b,)), ((), ()))
    dg = lambda a, b, c: lax.dot_general(a, b, c, preferred_element_type=f32)
    dot = lambda a, b: jnp.dot(a, b, preferred_element_type=f32)

    @pl.when(pl.program_id(0) == 0)
    def _prep():
        ei = ei_ref[...]
        ei0 = ei[0:1, :]
        ei1 = ei[1:2, :]
        niota = lax.broadcasted_iota(jnp.int32, (N, E), 0)
        ST = (ei0 == niota).astype(f32)
        DT = (ei1 == niota).astype(f32)

        deg = jnp.sum(DT, axis=1, keepdims=True) + 1.0
        dinv = lax.rsqrt(deg)

        dinv_src = dg(dinv, ST, cdim(0, 0))
        dinv_dst = dg(dinv, DT, cdim(0, 0))
        norm = dinv_src * dinv_dst

        AT = dg(ST * norm, DT, cdim(1, 1))
        ii = lax.broadcasted_iota(jnp.int32, (N, N), 0)
        jj = lax.broadcasted_iota(jnp.int32, (N, N), 1)
        AT = AT + jnp.where(ii == jj, dinv * dinv, 0.0)

        mi = lax.broadcasted_iota(jnp.int32, (N, NF), 0)
        ji = lax.broadcasted_iota(jnp.int32, (N, NF), 1)
        Ecol = (ji // F == mi).astype(f32)
        fi = lax.broadcasted_iota(jnp.int32, (F, NF), 0)
        gi = lax.broadcasted_iota(jnp.int32, (F, NF), 1)
        T4 = (gi % F == fi).astype(f32)

        R = dg(Ecol, AT, cdim(0, 1))
        w1c = dg(T4, w1_ref[...], cdim(0, 1))
        m1t_ref[...] = R * w1c

        ATeeT = dot(R, Ecol)
        U2 = dg(T4, w2_ref[...], cdim(0, 1))
        m2t_ref[...] = ATeeT * dot(U2, T4)
        U3 = dg(T4, w3_ref[...], cdim(0, 1))
        m3t_ref[...] = ATeeT * dot(U3, T4)

        b123c_ref[:, 0:1] = dg(T4, b1_ref[...], cdim(0, 1))
        b123c_ref[:, 1:2] = dg(T4, b2_ref[...], cdim(0, 1))
        b123c_ref[:, 2:3] = dg(T4, b3_ref[...], cdim(0, 1))

        i40a = lax.broadcasted_iota(jnp.int32, (NF, NF), 0)
        i40b = lax.broadcasted_iota(jnp.int32, (NF, NF), 1)
        I40 = (i40a == i40b).astype(f32)
        wl1t_ref[...] = dg(wl1_ref[...], I40, cdim(0, 0))
        i24a = lax.broadcasted_iota(jnp.int32, (H, H), 0)
        i24b = lax.broadcasted_iota(jnp.int32, (H, H), 1)
        I24 = (i24a == i24b).astype(f32)
        blc_ref[...] = dg(I24, bl1_ref[...], cdim(0, 1))
        wl2t_ref[...] = dg(wl2_ref[...], I24, cdim(0, 0))

    xT = x_ref[...]
    h1 = jnp.maximum(dot(m1t_ref[...], xT) + b123c_ref[:, 0:1], 0.0)
    h2 = jnp.maximum(dot(m2t_ref[...], h1) + b123c_ref[:, 1:2], 0.0)
    h3 = jnp.maximum(dot(m3t_ref[...], h2) + b123c_ref[:, 2:3] + h1, 0.0)
    z = jnp.maximum(dot(wl1t_ref[...], h3) + blc_ref[...], 0.0)
    out_ref[...] = dot(wl2t_ref[...], z) + bl2_ref[...]


def kernel(x1, edge_index, W1, b1, W2, b2, W3, b3, Wl1, bl1, Wl2, bl2):
    B = x1.shape[0]
    ei = edge_index.astype(jnp.int32)
    xT = x1.reshape(B, N).T

    f32 = jnp.float32
    full = lambda shape: pl.BlockSpec(shape, lambda i: tuple(0 for _ in shape))
    outT = pl.pallas_call(
        _fused_kernel,
        grid=(B // TL,),
        in_specs=[
            full((2, E)), full((1, F)), full((F, F)), full((F, F)),
            full((1, F)), full((1, F)), full((1, F)),
            full((NF, H)), full((1, H)), full((H, 1)), full((1, 1)),
            pl.BlockSpec((N, TL), lambda i: (0, i)),
        ],
        out_specs=pl.BlockSpec((1, TL), lambda i: (0, i)),
        out_shape=jax.ShapeDtypeStruct((1, B), f32),
        scratch_shapes=[
            pltpu.VMEM((NF, N), f32), pltpu.VMEM((NF, NF), f32),
            pltpu.VMEM((NF, NF), f32), pltpu.VMEM((NF, 3), f32),
            pltpu.VMEM((H, NF), f32), pltpu.VMEM((H, 1), f32),
            pltpu.VMEM((1, H), f32),
        ],
    )(ei, W1, W2, W3, b1[None, :], b2[None, :], b3[None, :],
      Wl1, bl1[None, :], Wl2, bl2[None, :], xT)
    return outT.reshape(B, 1)


# speedup vs baseline: 1.0859x; 1.0006x over previous
import jax
import jax.numpy as jnp
from jax import lax
from jax.experimental import pallas as pl
from jax.experimental.pallas import tpu as pltpu

N = 10
E = 30
F = 4
NF = N * F
H = 24
TL = 32768


def _fused_kernel(ei_ref, w1_ref, w2_ref, w3_ref, b1_ref, b2_ref, b3_ref,
                  wl1_ref, bl1_ref, wl2_ref, bl2_ref, x_ref, out_ref,
                  m1t_ref, m2t_ref, m3t_ref, b123c_ref,
                  wl1t_ref, blc_ref, wl2t_ref):
    f32 = jnp.float32
    cdim = lambda a, b: (((a,), (b,)), ((), ()))
    dg = lambda a, b, c: lax.dot_general(a, b, c, preferred_element_type=f32)
    dot = lambda a, b: jnp.dot(a, b, preferred_element_type=f32)

    @pl.when(pl.program_id(0) == 0)
    def _prep():
        ei = ei_ref[...]
        ei0 = ei[0:1, :]
        ei1 = ei[1:2, :]
        niota = lax.broadcasted_iota(jnp.int32, (N, E), 0)
        ST = (ei0 == niota).astype(f32)
        DT = (ei1 == niota).astype(f32)

        deg = jnp.sum(DT, axis=1, keepdims=True) + 1.0
        dinv = lax.rsqrt(deg)

        dinv_src = dg(dinv, ST, cdim(0, 0))
        dinv_dst = dg(dinv, DT, cdim(0, 0))
        norm = dinv_src * dinv_dst

        AT = dg(ST * norm, DT, cdim(1, 1))
        ii = lax.broadcasted_iota(jnp.int32, (N, N), 0)
        jj = lax.broadcasted_iota(jnp.int32, (N, N), 1)
        AT = AT + jnp.where(ii == jj, dinv * dinv, 0.0)

        mi = lax.broadcasted_iota(jnp.int32, (N, NF), 0)
        ji = lax.broadcasted_iota(jnp.int32, (N, NF), 1)
        Ecol = (ji // F == mi).astype(f32)
        fi = lax.broadcasted_iota(jnp.int32, (F, NF), 0)
        gi = lax.broadcasted_iota(jnp.int32, (F, NF), 1)
        T4 = (gi % F == fi).astype(f32)

        R = dg(Ecol, AT, cdim(0, 1))
        w1c = dg(T4, w1_ref[...], cdim(0, 1))
        m1t_ref[...] = R * w1c

        ATeeT = dot(R, Ecol)
        U2 = dg(T4, w2_ref[...], cdim(0, 1))
        m2t_ref[...] = ATeeT * dot(U2, T4)
        U3 = dg(T4, w3_ref[...], cdim(0, 1))
        m3t_ref[...] = ATeeT * dot(U3, T4)

        b123c_ref[:, 0:1] = dg(T4, b1_ref[...], cdim(0, 1))
        b123c_ref[:, 1:2] = dg(T4, b2_ref[...], cdim(0, 1))
        b123c_ref[:, 2:3] = dg(T4, b3_ref[...], cdim(0, 1))

        i40a = lax.broadcasted_iota(jnp.int32, (NF, NF), 0)
        i40b = lax.broadcasted_iota(jnp.int32, (NF, NF), 1)
        I40 = (i40a == i40b).astype(f32)
        wl1t_ref[...] = dg(wl1_ref[...], I40, cdim(0, 0))
        i24a = lax.broadcasted_iota(jnp.int32, (H, H), 0)
        i24b = lax.broadcasted_iota(jnp.int32, (H, H), 1)
        I24 = (i24a == i24b).astype(f32)
        blc_ref[...] = dg(I24, bl1_ref[...], cdim(0, 1))
        wl2t_ref[...] = dg(wl2_ref[...], I24, cdim(0, 0))

    xT = x_ref[...]
    h1 = jnp.maximum(dot(m1t_ref[...], xT) + b123c_ref[:, 0:1], 0.0)
    h2 = jnp.maximum(dot(m2t_ref[...], h1) + b123c_ref[:, 1:2], 0.0)
    h3 = jnp.maximum(dot(m3t_ref[...], h2) + b123c_ref[:, 2:3] + h1, 0.0)
    z = jnp.maximum(dot(wl1t_ref[...], h3) + blc_ref[...], 0.0)
    out_ref[...] = dot(wl2t_ref[...], z) + bl2_ref[...]


def kernel(x1, edge_index, W1, b1, W2, b2, W3, b3, Wl1, bl1, Wl2, bl2):
    B = x1.shape[0]
    ei = edge_index.astype(jnp.int32)
    xT = x1.reshape(B, N).T

    f32 = jnp.float32
    full = lambda shape: pl.BlockSpec(shape, lambda i: tuple(0 for _ in shape))
    outT = pl.pallas_call(
        _fused_kernel,
        grid=(B // TL,),
        in_specs=[
            full((2, E)), full((1, F)), full((F, F)), full((F, F)),
            full((1, F)), full((1, F)), full((1, F)),
            full((NF, H)), full((1, H)), full((H, 1)), full((1, 1)),
            pl.BlockSpec((N, TL), lambda i: (0, i)),
        ],
        out_specs=pl.BlockSpec((1, TL), lambda i: (0, i)),
        out_shape=jax.ShapeDtypeStruct((1, B), f32),
        scratch_shapes=[
            pltpu.VMEM((NF, N), f32), pltpu.VMEM((NF, NF), f32),
            pltpu.VMEM((NF, NF), f32), pltpu.VMEM((NF, 3), f32),
            pltpu.VMEM((H, NF), f32), pltpu.VMEM((H, 1), f32),
            pltpu.VMEM((1, H), f32),
        ],
    )(ei, W1, W2, W3, b1[None, :], b2[None, :], b3[None, :],
      Wl1, bl1[None, :], Wl2, bl2[None, :], xT)
    return outT.reshape(B, 1)


# speedup vs baseline: 1.4947x; 1.3766x over previous
import jax
import jax.numpy as jnp
from jax import lax
from jax.experimental import pallas as pl
from jax.experimental.pallas import tpu as pltpu

N = 10
E = 30
F = 4
NF = N * F
H = 24
TL = 32768


def _fused_kernel(ei_ref, w1_ref, w2_ref, w3_ref, b1_ref, b2_ref, b3_ref,
                  wl1_ref, bl1_ref, wl2_ref, bl2_ref, x_ref, out_ref,
                  m1t_ref, m2t_ref, m3t_ref, b123c_ref,
                  wl1t_ref, blc_ref, wl2t_ref):
    f32 = jnp.float32
    cdim = lambda a, b: (((a,), (b,)), ((), ()))
    dg = lambda a, b, c: lax.dot_general(a, b, c, preferred_element_type=f32)
    dot = lambda a, b: jnp.dot(a, b, preferred_element_type=f32)

    @pl.when(pl.program_id(0) == 0)
    def _prep():
        ei = ei_ref[...]
        ei0 = ei[0:1, :]
        ei1 = ei[1:2, :]
        niota = lax.broadcasted_iota(jnp.int32, (N, E), 0)
        ST = (ei0 == niota).astype(f32)
        DT = (ei1 == niota).astype(f32)

        deg = jnp.sum(DT, axis=1, keepdims=True) + 1.0
        dinv = lax.rsqrt(deg)

        dinv_src = dg(dinv, ST, cdim(0, 0))
        dinv_dst = dg(dinv, DT, cdim(0, 0))
        norm = dinv_src * dinv_dst

        AT = dg(ST * norm, DT, cdim(1, 1))
        ii = lax.broadcasted_iota(jnp.int32, (N, N), 0)
        jj = lax.broadcasted_iota(jnp.int32, (N, N), 1)
        AT = AT + jnp.where(ii == jj, dinv * dinv, 0.0)

        mi = lax.broadcasted_iota(jnp.int32, (N, NF), 0)
        ji = lax.broadcasted_iota(jnp.int32, (N, NF), 1)
        Ecol = (ji // F == mi).astype(f32)
        fi = lax.broadcasted_iota(jnp.int32, (F, NF), 0)
        gi = lax.broadcasted_iota(jnp.int32, (F, NF), 1)
        T4 = (gi % F == fi).astype(f32)

        R = dg(Ecol, AT, cdim(0, 1))
        w1c = dg(T4, w1_ref[...], cdim(0, 1))
        m1t_ref[...] = R * w1c

        ATeeT = dot(R, Ecol)
        U2 = dg(T4, w2_ref[...], cdim(0, 1))
        m2t_ref[...] = ATeeT * dot(U2, T4)
        U3 = dg(T4, w3_ref[...], cdim(0, 1))
        m3t_ref[...] = ATeeT * dot(U3, T4)

        b123c_ref[:, 0:1] = dg(T4, b1_ref[...], cdim(0, 1))
        b123c_ref[:, 1:2] = dg(T4, b2_ref[...], cdim(0, 1))
        b123c_ref[:, 2:3] = dg(T4, b3_ref[...], cdim(0, 1))

        i40a = lax.broadcasted_iota(jnp.int32, (NF, NF), 0)
        i40b = lax.broadcasted_iota(jnp.int32, (NF, NF), 1)
        I40 = (i40a == i40b).astype(f32)
        wl1t_ref[...] = dg(wl1_ref[...], I40, cdim(0, 0))
        i24a = lax.broadcasted_iota(jnp.int32, (H, H), 0)
        i24b = lax.broadcasted_iota(jnp.int32, (H, H), 1)
        I24 = (i24a == i24b).astype(f32)
        blc_ref[...] = dg(I24, bl1_ref[...], cdim(0, 1))
        wl2t_ref[...] = dg(wl2_ref[...], I24, cdim(0, 0))

    xT = x_ref[:, 0, :]
    h1 = jnp.maximum(dot(m1t_ref[...], xT) + b123c_ref[:, 0:1], 0.0)
    h2 = jnp.maximum(dot(m2t_ref[...], h1) + b123c_ref[:, 1:2], 0.0)
    h3 = jnp.maximum(dot(m3t_ref[...], h2) + b123c_ref[:, 2:3] + h1, 0.0)
    z = jnp.maximum(dot(wl1t_ref[...], h3) + blc_ref[...], 0.0)
    out_ref[...] = dot(wl2t_ref[...], z) + bl2_ref[...]


def kernel(x1, edge_index, W1, b1, W2, b2, W3, b3, Wl1, bl1, Wl2, bl2):
    B = x1.shape[0]
    ei = edge_index.astype(jnp.int32)
    x3 = jnp.transpose(x1, (1, 2, 0))

    f32 = jnp.float32
    full = lambda shape: pl.BlockSpec(shape, lambda i: tuple(0 for _ in shape))
    outT = pl.pallas_call(
        _fused_kernel,
        grid=(B // TL,),
        in_specs=[
            full((2, E)), full((1, F)), full((F, F)), full((F, F)),
            full((1, F)), full((1, F)), full((1, F)),
            full((NF, H)), full((1, H)), full((H, 1)), full((1, 1)),
            pl.BlockSpec((N, 1, TL), lambda i: (0, 0, i)),
        ],
        out_specs=pl.BlockSpec((1, TL), lambda i: (0, i)),
        out_shape=jax.ShapeDtypeStruct((1, B), f32),
        scratch_shapes=[
            pltpu.VMEM((NF, N), f32), pltpu.VMEM((NF, NF), f32),
            pltpu.VMEM((NF, NF), f32), pltpu.VMEM((NF, 3), f32),
            pltpu.VMEM((H, NF), f32), pltpu.VMEM((H, 1), f32),
            pltpu.VMEM((1, H), f32),
        ],
    )(ei, W1, W2, W3, b1[None, :], b2[None, :], b3[None, :],
      Wl1, bl1[None, :], Wl2, bl2[None, :], x3)
    return outT.reshape(B, 1)


# speedup vs baseline: 1.5313x; 1.0245x over previous
import jax
import jax.numpy as jnp
from jax import lax
from jax.experimental import pallas as pl
from jax.experimental.pallas import tpu as pltpu

N = 10
E = 30
F = 4
NF = N * F
H = 24
TL = 16384


def _fused_kernel(ei_ref, w1_ref, w2_ref, w3_ref, b1_ref, b2_ref, b3_ref,
                  wl1_ref, bl1_ref, wl2_ref, bl2_ref, x_ref, out_ref,
                  m1t_ref, m2t_ref, m3t_ref, b123c_ref,
                  wl1t_ref, blc_ref, wl2t_ref):
    f32 = jnp.float32
    cdim = lambda a, b: (((a,), (b,)), ((), ()))
    dg = lambda a, b, c: lax.dot_general(a, b, c, preferred_element_type=f32)
    dot = lambda a, b: jnp.dot(a, b, preferred_element_type=f32)

    @pl.when(pl.program_id(0) == 0)
    def _prep():
        ei = ei_ref[...]
        ei0 = ei[0:1, :]
        ei1 = ei[1:2, :]
        niota = lax.broadcasted_iota(jnp.int32, (N, E), 0)
        ST = (ei0 == niota).astype(f32)
        DT = (ei1 == niota).astype(f32)

        deg = jnp.sum(DT, axis=1, keepdims=True) + 1.0
        dinv = lax.rsqrt(deg)

        dinv_src = dg(dinv, ST, cdim(0, 0))
        dinv_dst = dg(dinv, DT, cdim(0, 0))
        norm = dinv_src * dinv_dst

        AT = dg(ST * norm, DT, cdim(1, 1))
        ii = lax.broadcasted_iota(jnp.int32, (N, N), 0)
        jj = lax.broadcasted_iota(jnp.int32, (N, N), 1)
        AT = AT + jnp.where(ii == jj, dinv * dinv, 0.0)

        mi = lax.broadcasted_iota(jnp.int32, (N, NF), 0)
        ji = lax.broadcasted_iota(jnp.int32, (N, NF), 1)
        Ecol = (ji // F == mi).astype(f32)
        fi = lax.broadcasted_iota(jnp.int32, (F, NF), 0)
        gi = lax.broadcasted_iota(jnp.int32, (F, NF), 1)
        T4 = (gi % F == fi).astype(f32)

        R = dg(Ecol, AT, cdim(0, 1))
        w1c = dg(T4, w1_ref[...], cdim(0, 1))
        m1t_ref[...] = R * w1c

        ATeeT = dot(R, Ecol)
        U2 = dg(T4, w2_ref[...], cdim(0, 1))
        m2t_ref[...] = ATeeT * dot(U2, T4)
        U3 = dg(T4, w3_ref[...], cdim(0, 1))
        m3t_ref[...] = ATeeT * dot(U3, T4)

        b123c_ref[:, 0:1] = dg(T4, b1_ref[...], cdim(0, 1))
        b123c_ref[:, 1:2] = dg(T4, b2_ref[...], cdim(0, 1))
        b123c_ref[:, 2:3] = dg(T4, b3_ref[...], cdim(0, 1))

        i40a = lax.broadcasted_iota(jnp.int32, (NF, NF), 0)
        i40b = lax.broadcasted_iota(jnp.int32, (NF, NF), 1)
        I40 = (i40a == i40b).astype(f32)
        wl1t_ref[...] = dg(wl1_ref[...], I40, cdim(0, 0))
        i24a = lax.broadcasted_iota(jnp.int32, (H, H), 0)
        i24b = lax.broadcasted_iota(jnp.int32, (H, H), 1)
        I24 = (i24a == i24b).astype(f32)
        blc_ref[...] = dg(I24, bl1_ref[...], cdim(0, 1))
        wl2t_ref[...] = dg(wl2_ref[...], I24, cdim(0, 0))

    xT = x_ref[:, 0, :]
    h1 = jnp.maximum(dot(m1t_ref[...], xT) + b123c_ref[:, 0:1], 0.0)
    h2 = jnp.maximum(dot(m2t_ref[...], h1) + b123c_ref[:, 1:2], 0.0)
    h3 = jnp.maximum(dot(m3t_ref[...], h2) + b123c_ref[:, 2:3] + h1, 0.0)
    z = jnp.maximum(dot(wl1t_ref[...], h3) + blc_ref[...], 0.0)
    out_ref[...] = dot(wl2t_ref[...], z) + bl2_ref[...]


def kernel(x1, edge_index, W1, b1, W2, b2, W3, b3, Wl1, bl1, Wl2, bl2):
    B = x1.shape[0]
    ei = edge_index.astype(jnp.int32)
    x3 = jnp.transpose(x1, (1, 2, 0))

    f32 = jnp.float32
    full = lambda shape: pl.BlockSpec(shape, lambda i: tuple(0 for _ in shape))
    outT = pl.pallas_call(
        _fused_kernel,
        grid=(B // TL,),
        in_specs=[
            full((2, E)), full((1, F)), full((F, F)), full((F, F)),
            full((1, F)), full((1, F)), full((1, F)),
            full((NF, H)), full((1, H)), full((H, 1)), full((1, 1)),
            pl.BlockSpec((N, 1, TL), lambda i: (0, 0, i)),
        ],
        out_specs=pl.BlockSpec((1, TL), lambda i: (0, i)),
        out_shape=jax.ShapeDtypeStruct((1, B), f32),
        scratch_shapes=[
            pltpu.VMEM((NF, N), f32), pltpu.VMEM((NF, NF), f32),
            pltpu.VMEM((NF, NF), f32), pltpu.VMEM((NF, 3), f32),
            pltpu.VMEM((H, NF), f32), pltpu.VMEM((H, 1), f32),
            pltpu.VMEM((1, H), f32),
        ],
    )(ei, W1, W2, W3, b1[None, :], b2[None, :], b3[None, :],
      Wl1, bl1[None, :], Wl2, bl2[None, :], x3)
    return outT.reshape(B, 1)
